# Initial kernel scaffold; baseline (speedup 1.0000x reference)
import jax
import jax.numpy as jnp
from jax.experimental import pallas as pl


def kernel(hidden_states, gate_w, gate_bias, w_gate_up, w_down, ws_gate_up, ws_down):
    raise NotImplementedError("write your pallas kernel here")


# speedup vs baseline: 1.4413x; 1.4413x over previous
import functools

import jax
import jax.numpy as jnp
from jax import lax
from jax.experimental import pallas as pl
from jax.experimental.pallas import tpu as pltpu
from jax.experimental.pallas import tpu_sc as plsc

T = 2048
D = 2048
E = 16
K = 2
FF = 1408
NG = 8
TG = 4
CAP = 512

TB = 256
NTB = T // TB
SLOTS = E * CAP
XS_ROWS = SLOTS + CAP
YS_ROWS = SLOTS + T
NSB = T // CAP
NRB = E + NSB
FT = 128
NF = FF // FT
RC = 128
NRC = CAP // RC

NC, NS, L = 2, 16, 16
NW = NC * NS
TPW = T // NW
CHT = 8
NCH = TPW // CHT

_NEG = -float("inf")


def _routing_body(x_ref, gw_ref, gb_ref, poss_ref, posg_ref, wts_ref,
                  cnt_ref, carry_ref):
    tb = pl.program_id(0)

    @pl.when(tb == 0)
    def _():
        carry_ref[...] = jnp.zeros_like(carry_ref)

    xb = x_ref[...]
    logits = lax.dot_general(xb, gw_ref[...], (((1,), (1,)), ((), ())),
                             preferred_element_type=jnp.float32)
    scores = jax.nn.sigmoid(logits)
    sfc = scores + gb_ref[...]

    ri = lax.broadcasted_iota(jnp.int32, (E, NG), 0)
    ci = lax.broadcasted_iota(jnp.int32, (E, NG), 1)
    m1 = (ri // (E // NG) == ci).astype(jnp.float32)

    gs = lax.dot_general(sfc, m1, (((1,), (0,)), ((), ())),
                         precision=lax.Precision.HIGHEST,
                         preferred_element_type=jnp.float32)
    lane8 = lax.broadcasted_iota(jnp.int32, (TB, NG), 1)
    gmask = jnp.zeros((TB, NG), jnp.float32)
    for _ in range(TG):
        m = jnp.max(gs, axis=1, keepdims=True)
        amax = jnp.min(jnp.where(gs == m, lane8, NG), axis=1, keepdims=True)
        pick = lane8 == amax
        gmask = jnp.where(pick, 1.0, gmask)
        gs = jnp.where(pick, _NEG, gs)
    mask16 = lax.dot_general(gmask, m1, (((1,), (1,)), ((), ())),
                             preferred_element_type=jnp.float32)

    masked = jnp.where(mask16 > 0, sfc, _NEG)
    lane16 = lax.broadcasted_iota(jnp.int32, (TB, E), 1)
    idxs, ws, picks = [], [], []
    for _ in range(K):
        m = jnp.max(masked, axis=1, keepdims=True)
        amax = jnp.min(jnp.where(masked == m, lane16, E), axis=1, keepdims=True)
        pick = lane16 == amax
        idxs.append(amax)
        ws.append(jnp.sum(jnp.where(pick, scores, 0.0), axis=1, keepdims=True))
        picks.append(pick)
        masked = jnp.where(pick, _NEG, masked)
    denom = ws[0] + ws[1] + 1e-20
    w0 = ws[0] / denom
    w1 = ws[1] / denom

    cnt = picks[0].astype(jnp.float32) + picks[1].astype(jnp.float32)
    rr = lax.broadcasted_iota(jnp.int32, (TB, TB), 0)
    cc = lax.broadcasted_iota(jnp.int32, (TB, TB), 1)
    ltri = (cc < rr).astype(jnp.float32)
    excl = lax.dot_general(ltri, cnt, (((1,), (0,)), ((), ())),
                           preferred_element_type=jnp.float32) + carry_ref[...]
    carry_ref[...] = carry_ref[...] + jnp.sum(cnt, axis=0, keepdims=True)

    rank0 = jnp.sum(jnp.where(picks[0], excl, 0.0), axis=1,
                    keepdims=True).astype(jnp.int32)
    rank1 = jnp.sum(jnp.where(picks[1], excl, 0.0), axis=1,
                    keepdims=True).astype(jnp.int32)
    base0 = idxs[0] * CAP
    base1 = idxs[1] * CAP
    ok0 = rank0 < CAP
    ok1 = rank1 < CAP
    poss_ref[...] = jnp.concatenate(
        [jnp.where(ok0, base0 + rank0, SLOTS),
         jnp.where(ok1, base1 + rank1, SLOTS)], axis=1)
    posg_ref[...] = jnp.concatenate(
        [jnp.where(ok0, base0 + rank0, base0),
         jnp.where(ok1, base1 + rank1, base1)], axis=1)
    wts_ref[...] = jnp.concatenate(
        [jnp.where(ok0, w0, 0.0), jnp.where(ok1, w1, 0.0)], axis=1)

    @pl.when(tb == NTB - 1)
    def _():
        cnt_ref[...] = jnp.minimum(carry_ref[...], float(CAP)).astype(jnp.int32)


def _route(x, gate_w, gb):
    return pl.pallas_call(
        _routing_body,
        grid=(NTB,),
        in_specs=[
            pl.BlockSpec((TB, D), lambda tb: (tb, 0)),
            pl.BlockSpec((E, D), lambda tb: (0, 0)),
            pl.BlockSpec((1, E), lambda tb: (0, 0)),
        ],
        out_specs=[
            pl.BlockSpec((TB, K), lambda tb: (tb, 0)),
            pl.BlockSpec((TB, K), lambda tb: (tb, 0)),
            pl.BlockSpec((TB, K), lambda tb: (tb, 0)),
            pl.BlockSpec((1, E), lambda tb: (0, 0)),
        ],
        out_shape=[
            jax.ShapeDtypeStruct((T, K), jnp.int32),
            jax.ShapeDtypeStruct((T, K), jnp.int32),
            jax.ShapeDtypeStruct((T, K), jnp.float32),
            jax.ShapeDtypeStruct((1, E), jnp.int32),
        ],
        scratch_shapes=[pltpu.VMEM((1, E), jnp.float32)],
        compiler_params=pltpu.CompilerParams(
            dimension_semantics=("arbitrary",)),
    )(x, gate_w, gb)


def _mlp_body(cnts_ref, xs_ref, x_ref, wgu_ref, wdn_ref, wgus_ref, wdns_ref,
              ys_ref, acc_ref):
    rb = pl.program_id(0)
    ft = pl.program_id(1)

    def compute(x_in, wgu_in, wdn_in):
        nrows = jnp.where(rb < E, cnts_ref[0, jnp.minimum(rb, E - 1)], CAP)
        wg = wgu_in[0, 0]
        wu = wgu_in[0, 1]
        wd = wdn_in[0]
        for rc in range(NRC):
            @pl.when(rc * RC < nrows)
            def _():
                xv = x_in[rc * RC:(rc + 1) * RC, :]
                g = lax.dot_general(xv, wg, (((1,), (1,)), ((), ())),
                                    preferred_element_type=jnp.float32)
                u = lax.dot_general(xv, wu, (((1,), (1,)), ((), ())),
                                    preferred_element_type=jnp.float32)
                h = (g * jax.nn.sigmoid(g)) * u
                d = lax.dot_general(h, wd, (((1,), (1,)), ((), ())),
                                    preferred_element_type=jnp.float32)
                sl = slice(rc * RC, (rc + 1) * RC)

                @pl.when(ft == 0)
                def _():
                    acc_ref[sl, :] = d

                @pl.when(ft > 0)
                def _():
                    acc_ref[sl, :] = acc_ref[sl, :] + d

    @pl.when(rb < E)
    def _():
        compute(xs_ref, wgu_ref, wdn_ref)

    @pl.when(rb >= E)
    def _():
        compute(x_ref, wgus_ref, wdns_ref)

    @pl.when(ft == NF - 1)
    def _():
        ys_ref[...] = acc_ref[...]


def _mlp(cnts, xs, x, wgu, wdn, wgus, wdns):
    grid_spec = pltpu.PrefetchScalarGridSpec(
        num_scalar_prefetch=1,
        grid=(NRB, NF),
        in_specs=[
            pl.BlockSpec((CAP, D), lambda rb, ft, c: (jnp.minimum(rb, E - 1), 0)),
            pl.BlockSpec((CAP, D), lambda rb, ft, c: (jnp.maximum(rb - E, 0), 0)),
            pl.BlockSpec((1, 2, FT, D),
                         lambda rb, ft, c: (jnp.minimum(rb, E - 1), 0,
                                            jnp.where(rb >= E, NF - 1, ft), 0)),
            pl.BlockSpec((1, D, FT),
                         lambda rb, ft, c: (jnp.minimum(rb, E - 1), 0,
                                            jnp.where(rb >= E, NF - 1, ft))),
            pl.BlockSpec((1, 2, FT, D),
                         lambda rb, ft, c: (0, 0, jnp.where(rb < E, 0, ft), 0)),
            pl.BlockSpec((1, D, FT),
                         lambda rb, ft, c: (0, 0, jnp.where(rb < E, 0, ft))),
        ],
        out_specs=pl.BlockSpec((CAP, D), lambda rb, ft, c: (rb, 0)),
        scratch_shapes=[pltpu.VMEM((CAP, D), jnp.float32)],
    )
    return pl.pallas_call(
        _mlp_body,
        grid_spec=grid_spec,
        out_shape=jax.ShapeDtypeStruct((YS_ROWS, D), jnp.float32),
        compiler_params=pltpu.CompilerParams(
            dimension_semantics=("arbitrary", "arbitrary"),
            vmem_limit_bytes=100 * 1024 * 1024),
    )(cnts, xs, x, wgu, wdn, wgus, wdns)


@functools.cache
def _sc_kernels():
    mesh = plsc.VectorSubcoreMesh(core_axis_name="c", subcore_axis_name="s",
                                  num_cores=NC, num_subcores=NS)

    @functools.partial(
        pl.kernel,
        mesh=mesh,
        out_type=jax.ShapeDtypeStruct((XS_ROWS, D), jnp.float32),
        scratch_types=[
            pltpu.VMEM((K * CHT,), jnp.int32),
            pltpu.VMEM((K * CHT,), jnp.int32),
            pltpu.VMEM((K * CHT, D), jnp.float32),
            pltpu.SemaphoreType.DMA,
        ],
    )
    def _dispatch(x_hbm, poss_hbm, xs_hbm, dup_v, pos_v, rows_v, sem):
        wid = lax.axis_index("s") * NC + lax.axis_index("c")
        t0 = wid * TPW
        lane = lax.iota(jnp.int32, L)

        def chunk(ci, _):
            tc0 = t0 + ci * CHT
            dup_v[...] = tc0 + lax.shift_right_logical(lane, 1)
            pltpu.sync_copy(poss_hbm.at[pl.ds(tc0 * K, K * CHT)], pos_v)
            pltpu.async_copy(x_hbm.at[dup_v], rows_v, sem).wait()
            pltpu.async_copy(rows_v, xs_hbm.at[pos_v], sem).wait()
            return 0

        lax.fori_loop(0, NCH, chunk, 0)

    @functools.partial(
        pl.kernel,
        mesh=mesh,
        out_type=jax.ShapeDtypeStruct((T, D), jnp.float32),
        scratch_types=[
            pltpu.VMEM((K * CHT,), jnp.int32),
            pltpu.VMEM((K * CHT,), jnp.float32),
            pltpu.VMEM((K * CHT, D), jnp.float32),
            pltpu.VMEM((CHT, D), jnp.float32),
            pltpu.VMEM((CHT, D), jnp.float32),
            pltpu.SemaphoreType.DMA,
        ],
        compiler_params=pltpu.CompilerParams(needs_layout_passes=False),
    )
    def _combine(ys_hbm, posg_hbm, wts_hbm, out_hbm, pos_v, w_v, rows_v, sh_v,
                 o_v, sem):
        wid = lax.axis_index("s") * NC + lax.axis_index("c")
        t0 = wid * TPW

        zero16 = lax.iota(jnp.int32, L) * 0

        def chunk(ci, _):
            tc0 = t0 + ci * CHT
            pltpu.sync_copy(posg_hbm.at[pl.ds(tc0 * K, K * CHT)], pos_v)
            pltpu.sync_copy(wts_hbm.at[pl.ds(tc0 * K, K * CHT)], w_v)
            pltpu.async_copy(ys_hbm.at[pos_v], rows_v, sem).wait()
            pltpu.sync_copy(ys_hbm.at[pl.ds(SLOTS + tc0, CHT)], sh_v)
            for j in range(CHT):
                w1 = plsc.load_gather(w_v, [zero16 + (K * j + 1)])
                if j == 0:
                    w0 = 1.0 - w1
                else:
                    w0 = plsc.load_gather(w_v, [zero16 + (K * j)])

                def col(v, _):
                    sl = pl.ds(v * L, L)
                    o_v[j, sl] = (sh_v[j, sl] + w0 * rows_v[K * j, sl]
                                  + w1 * rows_v[K * j + 1, sl])
                    return 0

                lax.fori_loop(0, D // L, col, 0)
            pltpu.sync_copy(o_v, out_hbm.at[pl.ds(tc0, CHT)])
            return 0

        lax.fori_loop(0, NCH, chunk, 0)

    return _dispatch, _combine


def kernel(hidden_states, gate_w, gate_bias, w_gate_up, w_down, ws_gate_up,
           ws_down):
    x = hidden_states
    gb = gate_bias.reshape(1, E)
    wgu = w_gate_up.reshape(E, 2, FF, D)
    wgus = ws_gate_up.reshape(1, 2, FF, D)
    wdn = w_down.reshape(E, D, FF)
    wdns = ws_down.reshape(1, D, FF)

    dispatch, combine = _sc_kernels()
    poss, posg, wts, cnts = _route(x, gate_w, gb)
    xs = dispatch(x, poss.reshape(T * K))
    ys = _mlp(cnts.reshape(1, E), xs, x, wgu, wdn, wgus, wdns)
    out = combine(ys, posg.reshape(T * K), wts.reshape(T * K))
    return out


# speedup vs baseline: 1.7410x; 1.2079x over previous
import functools

import jax
import jax.numpy as jnp
from jax import lax
from jax.experimental import pallas as pl
from jax.experimental.pallas import tpu as pltpu
from jax.experimental.pallas import tpu_sc as plsc

T = 2048
D = 2048
E = 16
K = 2
FF = 1408
NG = 8
TG = 4
CAP = 512

TB = 256
NTB = T // TB
SLOTS = E * CAP
XS_ROWS = SLOTS + CAP
YS_ROWS = SLOTS + T
NSB = T // CAP
NRB = E + NSB
FT = 128
NF = FF // FT
RC = 128
NRC = CAP // RC

NC, NS, L = 2, 16, 16
NW = NC * NS
TPW = T // NW
CHT = 8
NCH = TPW // CHT

_NEG = -float("inf")


def _routing_body(x_ref, gw_ref, gb_ref, poss_ref, posg_ref, wts_ref,
                  cnt_ref, carry_ref):
    tb = pl.program_id(0)

    @pl.when(tb == 0)
    def _():
        carry_ref[...] = jnp.zeros_like(carry_ref)

    xb = x_ref[...]
    logits = lax.dot_general(xb, gw_ref[...], (((1,), (1,)), ((), ())),
                             preferred_element_type=jnp.float32)
    scores = jax.nn.sigmoid(logits)
    sfc = scores + gb_ref[...]

    ri = lax.broadcasted_iota(jnp.int32, (E, NG), 0)
    ci = lax.broadcasted_iota(jnp.int32, (E, NG), 1)
    m1 = (ri // (E // NG) == ci).astype(jnp.float32)

    gs = lax.dot_general(sfc, m1, (((1,), (0,)), ((), ())),
                         precision=lax.Precision.HIGHEST,
                         preferred_element_type=jnp.float32)
    lane8 = lax.broadcasted_iota(jnp.int32, (TB, NG), 1)
    gmask = jnp.zeros((TB, NG), jnp.float32)
    for _ in range(TG):
        m = jnp.max(gs, axis=1, keepdims=True)
        amax = jnp.min(jnp.where(gs == m, lane8, NG), axis=1, keepdims=True)
        pick = lane8 == amax
        gmask = jnp.where(pick, 1.0, gmask)
        gs = jnp.where(pick, _NEG, gs)
    mask16 = lax.dot_general(gmask, m1, (((1,), (1,)), ((), ())),
                             preferred_element_type=jnp.float32)

    masked = jnp.where(mask16 > 0, sfc, _NEG)
    lane16 = lax.broadcasted_iota(jnp.int32, (TB, E), 1)
    idxs, ws, picks = [], [], []
    for _ in range(K):
        m = jnp.max(masked, axis=1, keepdims=True)
        amax = jnp.min(jnp.where(masked == m, lane16, E), axis=1, keepdims=True)
        pick = lane16 == amax
        idxs.append(amax)
        ws.append(jnp.sum(jnp.where(pick, scores, 0.0), axis=1, keepdims=True))
        picks.append(pick)
        masked = jnp.where(pick, _NEG, masked)
    denom = ws[0] + ws[1] + 1e-20
    w0 = ws[0] / denom
    w1 = ws[1] / denom

    cnt = picks[0].astype(jnp.float32) + picks[1].astype(jnp.float32)
    rr = lax.broadcasted_iota(jnp.int32, (TB, TB), 0)
    cc = lax.broadcasted_iota(jnp.int32, (TB, TB), 1)
    ltri = (cc < rr).astype(jnp.float32)
    excl = lax.dot_general(ltri, cnt, (((1,), (0,)), ((), ())),
                           preferred_element_type=jnp.float32) + carry_ref[...]
    carry_ref[...] = carry_ref[...] + jnp.sum(cnt, axis=0, keepdims=True)

    rank0 = jnp.sum(jnp.where(picks[0], excl, 0.0), axis=1,
                    keepdims=True).astype(jnp.int32)
    rank1 = jnp.sum(jnp.where(picks[1], excl, 0.0), axis=1,
                    keepdims=True).astype(jnp.int32)
    base0 = idxs[0] * CAP
    base1 = idxs[1] * CAP
    ok0 = rank0 < CAP
    ok1 = rank1 < CAP
    poss_ref[...] = jnp.concatenate(
        [jnp.where(ok0, base0 + rank0, SLOTS),
         jnp.where(ok1, base1 + rank1, SLOTS)], axis=1)
    posg_ref[...] = jnp.concatenate(
        [jnp.where(ok0, base0 + rank0, base0),
         jnp.where(ok1, base1 + rank1, base1)], axis=1)
    wts_ref[...] = jnp.concatenate(
        [jnp.where(ok0, w0, 0.0), jnp.where(ok1, w1, 0.0)], axis=1)

    @pl.when(tb == NTB - 1)
    def _():
        cnt_ref[...] = jnp.minimum(carry_ref[...], float(CAP)).astype(jnp.int32)


def _route(x, gate_w, gb):
    return pl.pallas_call(
        _routing_body,
        grid=(NTB,),
        in_specs=[
            pl.BlockSpec((TB, D), lambda tb: (tb, 0)),
            pl.BlockSpec((E, D), lambda tb: (0, 0)),
            pl.BlockSpec((1, E), lambda tb: (0, 0)),
        ],
        out_specs=[
            pl.BlockSpec((TB, K), lambda tb: (tb, 0)),
            pl.BlockSpec((TB, K), lambda tb: (tb, 0)),
            pl.BlockSpec((TB, K), lambda tb: (tb, 0)),
            pl.BlockSpec((1, E), lambda tb: (0, 0)),
        ],
        out_shape=[
            jax.ShapeDtypeStruct((T, K), jnp.int32),
            jax.ShapeDtypeStruct((T, K), jnp.int32),
            jax.ShapeDtypeStruct((T, K), jnp.float32),
            jax.ShapeDtypeStruct((1, E), jnp.int32),
        ],
        scratch_shapes=[pltpu.VMEM((1, E), jnp.float32)],
        compiler_params=pltpu.CompilerParams(
            dimension_semantics=("arbitrary",)),
    )(x, gate_w, gb)


def _ffn_step(nrows, x_in, wgu_in, wdn_in, ys_ref, h_ref, ft):
    def when_active(rc, fn):
        if nrows is None:
            fn()
        else:
            pl.when(rc * RC < nrows)(fn)

    wboth = wgu_in[...].reshape(2 * FT, D)
    for rc in range(NRC):
        sl = slice(rc * RC, (rc + 1) * RC)

        def gu_step(sl=sl):
            xv = x_in[sl, :]
            gu = lax.dot_general(xv, wboth, (((1,), (1,)), ((), ())),
                                 preferred_element_type=jnp.float32)
            g = gu[:, :FT]
            u = gu[:, FT:]
            h_ref[sl, pl.ds(ft * FT, FT)] = (g * jax.nn.sigmoid(g)) * u

        when_active(rc, gu_step)

    @pl.when(ft == NF - 1)
    def _():
        wd = wdn_in[0]
        for rc in range(NRC):
            sl = slice(rc * RC, (rc + 1) * RC)

            def dn_step(sl=sl):
                ys_ref[sl, :] = lax.dot_general(
                    h_ref[sl, :], wd, (((1,), (1,)), ((), ())),
                    preferred_element_type=jnp.float32)

            when_active(rc, dn_step)


def _mlp_body(cnts_ref, xs_ref, wgu_ref, wdn_ref, ys_ref, h_ref):
    rb = pl.program_id(0)
    ft = pl.program_id(1)
    nrows = cnts_ref[0, rb]
    _ffn_step(nrows, xs_ref, wgu_ref, wdn_ref, ys_ref, h_ref, ft)


def _mlp(cnts, xs, wgu, wdn):
    grid_spec = pltpu.PrefetchScalarGridSpec(
        num_scalar_prefetch=1,
        grid=(E, NF),
        in_specs=[
            pl.BlockSpec((CAP, D), lambda rb, ft, c: (rb, 0)),
            pl.BlockSpec((1, 2, FT, D), lambda rb, ft, c: (rb, 0, ft, 0)),
            pl.BlockSpec((1, D, FF), lambda rb, ft, c: (rb, 0, 0)),
        ],
        out_specs=pl.BlockSpec((CAP, D), lambda rb, ft, c: (rb, 0)),
        scratch_shapes=[pltpu.VMEM((CAP, FF), jnp.float32)],
    )
    return pl.pallas_call(
        _mlp_body,
        grid_spec=grid_spec,
        out_shape=jax.ShapeDtypeStruct((SLOTS, D), jnp.float32),
        compiler_params=pltpu.CompilerParams(
            dimension_semantics=("arbitrary", "arbitrary"),
            vmem_limit_bytes=60 * 1024 * 1024),
    )(cnts, xs, wgu, wdn)


def _shared_body(x_ref, wgu_ref, wdn_ref, ys_ref, h_ref):
    ft = pl.program_id(1)
    _ffn_step(None, x_ref, wgu_ref, wdn_ref, ys_ref, h_ref, ft)


def _shared_mlp(x, wgus, wdns):
    return pl.pallas_call(
        _shared_body,
        grid=(NSB, NF),
        in_specs=[
            pl.BlockSpec((CAP, D), lambda sb, ft: (sb, 0)),
            pl.BlockSpec((1, 2, FT, D), lambda sb, ft: (0, 0, ft, 0)),
            pl.BlockSpec((1, D, FF), lambda sb, ft: (0, 0, 0)),
        ],
        out_specs=pl.BlockSpec((CAP, D), lambda sb, ft: (sb, 0)),
        out_shape=jax.ShapeDtypeStruct((T, D), jnp.float32),
        scratch_shapes=[pltpu.VMEM((CAP, FF), jnp.float32)],
        compiler_params=pltpu.CompilerParams(
            dimension_semantics=("arbitrary", "arbitrary"),
            vmem_limit_bytes=60 * 1024 * 1024),
    )(x, wgus, wdns)


@functools.cache
def _sc_kernels():
    mesh = plsc.VectorSubcoreMesh(core_axis_name="c", subcore_axis_name="s",
                                  num_cores=NC, num_subcores=NS)

    @functools.partial(
        pl.kernel,
        mesh=mesh,
        out_type=jax.ShapeDtypeStruct((XS_ROWS, D), jnp.float32),
        scratch_types=[
            pltpu.VMEM((K * CHT,), jnp.int32),
            pltpu.VMEM((K * CHT,), jnp.int32),
            pltpu.VMEM((K * CHT, D), jnp.float32),
            pltpu.SemaphoreType.DMA,
        ],
    )
    def _dispatch(x_hbm, poss_hbm, xs_hbm, dup_v, pos_v, rows_v, sem):
        wid = lax.axis_index("s") * NC + lax.axis_index("c")
        t0 = wid * TPW
        lane = lax.iota(jnp.int32, L)

        def chunk(ci, _):
            tc0 = t0 + ci * CHT
            dup_v[...] = tc0 + lax.shift_right_logical(lane, 1)
            pltpu.sync_copy(poss_hbm.at[pl.ds(tc0 * K, K * CHT)], pos_v)
            pltpu.async_copy(x_hbm.at[dup_v], rows_v, sem).wait()
            pltpu.async_copy(rows_v, xs_hbm.at[pos_v], sem).wait()
            return 0

        lax.fori_loop(0, NCH, chunk, 0)

    @functools.partial(
        pl.kernel,
        mesh=mesh,
        out_type=jax.ShapeDtypeStruct((T, D), jnp.float32),
        scratch_types=[
            pltpu.VMEM((K * CHT,), jnp.int32),
            pltpu.VMEM((K * CHT,), jnp.float32),
            pltpu.VMEM((K * CHT, D), jnp.float32),
            pltpu.VMEM((CHT, D), jnp.float32),
            pltpu.VMEM((CHT, D), jnp.float32),
            pltpu.SemaphoreType.DMA,
        ],
        compiler_params=pltpu.CompilerParams(needs_layout_passes=False),
    )
    def _combine(ys_hbm, yssh_hbm, posg_hbm, wts_hbm, out_hbm, pos_v, w_v,
                 rows_v, sh_v, o_v, sem):
        wid = lax.axis_index("s") * NC + lax.axis_index("c")
        t0 = wid * TPW

        zero16 = lax.iota(jnp.int32, L) * 0

        def chunk(ci, _):
            tc0 = t0 + ci * CHT
            pltpu.sync_copy(posg_hbm.at[pl.ds(tc0 * K, K * CHT)], pos_v)
            pltpu.sync_copy(wts_hbm.at[pl.ds(tc0 * K, K * CHT)], w_v)
            pltpu.async_copy(ys_hbm.at[pos_v], rows_v, sem).wait()
            pltpu.sync_copy(yssh_hbm.at[pl.ds(tc0, CHT)], sh_v)
            for j in range(CHT):
                w1 = plsc.load_gather(w_v, [zero16 + (K * j + 1)])
                if j == 0:
                    w0 = 1.0 - w1
                else:
                    w0 = plsc.load_gather(w_v, [zero16 + (K * j)])

                def col(v, _):
                    sl = pl.ds(v * L, L)
                    o_v[j, sl] = (sh_v[j, sl] + w0 * rows_v[K * j, sl]
                                  + w1 * rows_v[K * j + 1, sl])
                    return 0

                lax.fori_loop(0, D // L, col, 0)
            pltpu.sync_copy(o_v, out_hbm.at[pl.ds(tc0, CHT)])
            return 0

        lax.fori_loop(0, NCH, chunk, 0)

    return _dispatch, _combine


def kernel(hidden_states, gate_w, gate_bias, w_gate_up, w_down, ws_gate_up,
           ws_down):
    x = hidden_states
    gb = gate_bias.reshape(1, E)
    wgu = w_gate_up.reshape(E, 2, FF, D)
    wgus = ws_gate_up.reshape(1, 2, FF, D)
    wdn = w_down.reshape(E, D, FF)
    wdns = ws_down.reshape(1, D, FF)

    dispatch, combine = _sc_kernels()
    poss, posg, wts, cnts = _route(x, gate_w, gb)
    xs = dispatch(x, poss.reshape(T * K))
    ys_sh = _shared_mlp(x, wgus, wdns)
    ys = _mlp(cnts, xs, wgu, wdn)
    out = combine(ys, ys_sh, posg.reshape(T * K), wts.reshape(T * K))
    return out


# speedup vs baseline: 1.7826x; 1.0239x over previous
import functools

import jax
import jax.numpy as jnp
from jax import lax
from jax.experimental import pallas as pl
from jax.experimental.pallas import tpu as pltpu
from jax.experimental.pallas import tpu_sc as plsc

T = 2048
D = 2048
E = 16
K = 2
FF = 1408
NG = 8
TG = 4
CAP = 512

TB = 256
NTB = T // TB
SLOTS = E * CAP
XS_ROWS = SLOTS + CAP
YS_ROWS = SLOTS + T
NSB = T // CAP
NRB = E + NSB
FT = 128
NF = FF // FT
RC = 128
NRC = CAP // RC

NC, NS, L = 2, 16, 16
NW = NC * NS
TPW = T // NW
CHT = 8
NCH = TPW // CHT

_NEG = -float("inf")


def _routing_body(x_ref, gw_ref, gb_ref, poss_ref, posg_ref, wts_ref,
                  cnt_ref, carry_ref):
    tb = pl.program_id(0)

    @pl.when(tb == 0)
    def _():
        carry_ref[...] = jnp.zeros_like(carry_ref)

    xb = x_ref[...]
    logits = lax.dot_general(xb, gw_ref[...], (((1,), (1,)), ((), ())),
                             preferred_element_type=jnp.float32)
    scores = jax.nn.sigmoid(logits)
    sfc = scores + gb_ref[...]

    ri = lax.broadcasted_iota(jnp.int32, (E, NG), 0)
    ci = lax.broadcasted_iota(jnp.int32, (E, NG), 1)
    m1 = (ri // (E // NG) == ci).astype(jnp.float32)

    gs = lax.dot_general(sfc, m1, (((1,), (0,)), ((), ())),
                         precision=lax.Precision.HIGHEST,
                         preferred_element_type=jnp.float32)
    lane8 = lax.broadcasted_iota(jnp.int32, (TB, NG), 1)
    gmask = jnp.zeros((TB, NG), jnp.float32)
    for _ in range(TG):
        m = jnp.max(gs, axis=1, keepdims=True)
        amax = jnp.min(jnp.where(gs == m, lane8, NG), axis=1, keepdims=True)
        pick = lane8 == amax
        gmask = jnp.where(pick, 1.0, gmask)
        gs = jnp.where(pick, _NEG, gs)
    mask16 = lax.dot_general(gmask, m1, (((1,), (1,)), ((), ())),
                             preferred_element_type=jnp.float32)

    masked = jnp.where(mask16 > 0, sfc, _NEG)
    lane16 = lax.broadcasted_iota(jnp.int32, (TB, E), 1)
    idxs, ws, picks = [], [], []
    for _ in range(K):
        m = jnp.max(masked, axis=1, keepdims=True)
        amax = jnp.min(jnp.where(masked == m, lane16, E), axis=1, keepdims=True)
        pick = lane16 == amax
        idxs.append(amax)
        ws.append(jnp.sum(jnp.where(pick, scores, 0.0), axis=1, keepdims=True))
        picks.append(pick)
        masked = jnp.where(pick, _NEG, masked)
    denom = ws[0] + ws[1] + 1e-20
    w0 = ws[0] / denom
    w1 = ws[1] / denom

    cnt = picks[0].astype(jnp.float32) + picks[1].astype(jnp.float32)
    rr = lax.broadcasted_iota(jnp.int32, (TB, TB), 0)
    cc = lax.broadcasted_iota(jnp.int32, (TB, TB), 1)
    ltri = (cc < rr).astype(jnp.float32)
    excl = lax.dot_general(ltri, cnt, (((1,), (0,)), ((), ())),
                           preferred_element_type=jnp.float32) + carry_ref[...]
    carry_ref[...] = carry_ref[...] + jnp.sum(cnt, axis=0, keepdims=True)

    rank0 = jnp.sum(jnp.where(picks[0], excl, 0.0), axis=1,
                    keepdims=True).astype(jnp.int32)
    rank1 = jnp.sum(jnp.where(picks[1], excl, 0.0), axis=1,
                    keepdims=True).astype(jnp.int32)
    base0 = idxs[0] * CAP
    base1 = idxs[1] * CAP
    ok0 = rank0 < CAP
    ok1 = rank1 < CAP
    poss_ref[...] = jnp.concatenate(
        [jnp.where(ok0, base0 + rank0, SLOTS),
         jnp.where(ok1, base1 + rank1, SLOTS)], axis=1)
    posg_ref[...] = jnp.concatenate(
        [jnp.where(ok0, base0 + rank0, base0),
         jnp.where(ok1, base1 + rank1, base1)], axis=1)
    wts_ref[...] = jnp.concatenate(
        [jnp.where(ok0, w0, 0.0), jnp.where(ok1, w1, 0.0)], axis=1)

    @pl.when(tb == NTB - 1)
    def _():
        cnt_ref[...] = jnp.minimum(carry_ref[...], float(CAP)).astype(jnp.int32)


def _route(x, gate_w, gb):
    return pl.pallas_call(
        _routing_body,
        grid=(NTB,),
        in_specs=[
            pl.BlockSpec((TB, D), lambda tb: (tb, 0)),
            pl.BlockSpec((E, D), lambda tb: (0, 0)),
            pl.BlockSpec((1, E), lambda tb: (0, 0)),
        ],
        out_specs=[
            pl.BlockSpec((TB, K), lambda tb: (tb, 0)),
            pl.BlockSpec((TB, K), lambda tb: (tb, 0)),
            pl.BlockSpec((TB, K), lambda tb: (tb, 0)),
            pl.BlockSpec((1, E), lambda tb: (0, 0)),
        ],
        out_shape=[
            jax.ShapeDtypeStruct((T, K), jnp.int32),
            jax.ShapeDtypeStruct((T, K), jnp.int32),
            jax.ShapeDtypeStruct((T, K), jnp.float32),
            jax.ShapeDtypeStruct((1, E), jnp.int32),
        ],
        scratch_shapes=[pltpu.VMEM((1, E), jnp.float32)],
        compiler_params=pltpu.CompilerParams(
            dimension_semantics=("arbitrary",)),
    )(x, gate_w, gb)


def _ffn_step(nrows, x_in, wgu_in, wdn_in, ys_ref, h_ref, ft, nrc=NRC):
    def when_active(rc, fn):
        if nrows is None:
            fn()
        else:
            pl.when(rc * RC < nrows)(fn)

    wboth = wgu_in[...].reshape(2 * FT, D)
    for rc in range(nrc):
        sl = slice(rc * RC, (rc + 1) * RC)

        def gu_step(sl=sl):
            xv = x_in[sl, :]
            gu = lax.dot_general(xv, wboth, (((1,), (1,)), ((), ())),
                                 preferred_element_type=jnp.float32)
            g = gu[:, :FT]
            u = gu[:, FT:]
            h_ref[sl, pl.ds(ft * FT, FT)] = (g * jax.nn.sigmoid(g)) * u

        when_active(rc, gu_step)

    @pl.when(ft == NF - 1)
    def _():
        wd = wdn_in[0]
        for rc in range(nrc):
            sl = slice(rc * RC, (rc + 1) * RC)

            def dn_step(sl=sl):
                ys_ref[sl, :] = lax.dot_general(
                    h_ref[sl, :], wd, (((1,), (1,)), ((), ())),
                    preferred_element_type=jnp.float32)

            when_active(rc, dn_step)


def _mlp_body(cnts_ref, xs_ref, wgu_ref, wdn_ref, ys_ref, h_ref):
    rb = pl.program_id(0)
    ft = pl.program_id(1)
    nrows = cnts_ref[0, rb]
    _ffn_step(nrows, xs_ref, wgu_ref, wdn_ref, ys_ref, h_ref, ft)


def _mlp(cnts, xs, wgu, wdn):
    grid_spec = pltpu.PrefetchScalarGridSpec(
        num_scalar_prefetch=1,
        grid=(E, NF),
        in_specs=[
            pl.BlockSpec((CAP, D), lambda rb, ft, c: (rb, 0)),
            pl.BlockSpec((1, 2, FT, D), lambda rb, ft, c: (rb, 0, ft, 0)),
            pl.BlockSpec((1, D, FF), lambda rb, ft, c: (rb, 0, 0)),
        ],
        out_specs=pl.BlockSpec((CAP, D), lambda rb, ft, c: (rb, 0)),
        scratch_shapes=[pltpu.VMEM((CAP, FF), jnp.float32)],
    )
    return pl.pallas_call(
        _mlp_body,
        grid_spec=grid_spec,
        out_shape=jax.ShapeDtypeStruct((SLOTS, D), jnp.float32),
        compiler_params=pltpu.CompilerParams(
            dimension_semantics=("arbitrary", "arbitrary"),
            vmem_limit_bytes=60 * 1024 * 1024),
    )(cnts, xs, wgu, wdn)


SHB = T // 2


def _shared_body(x_ref, wgu_ref, wdn_ref, ys_ref, h_ref):
    ft = pl.program_id(1)
    _ffn_step(None, x_ref, wgu_ref, wdn_ref, ys_ref, h_ref, ft, nrc=SHB // RC)


def _shared_mlp(x, wgus, wdns):
    return pl.pallas_call(
        _shared_body,
        grid=(2, NF),
        in_specs=[
            pl.BlockSpec((SHB, D), lambda sb, ft: (sb, 0)),
            pl.BlockSpec((1, 2, FT, D), lambda sb, ft: (0, 0, ft, 0)),
            pl.BlockSpec((1, D, FF), lambda sb, ft: (0, 0, 0)),
        ],
        out_specs=pl.BlockSpec((SHB, D), lambda sb, ft: (sb, 0)),
        out_shape=jax.ShapeDtypeStruct((T, D), jnp.float32),
        scratch_shapes=[pltpu.VMEM((SHB, FF), jnp.float32)],
        compiler_params=pltpu.CompilerParams(
            dimension_semantics=("arbitrary", "arbitrary"),
            vmem_limit_bytes=60 * 1024 * 1024),
    )(x, wgus, wdns)


@functools.cache
def _sc_kernels():
    mesh = plsc.VectorSubcoreMesh(core_axis_name="c", subcore_axis_name="s",
                                  num_cores=NC, num_subcores=NS)

    @functools.partial(
        pl.kernel,
        mesh=mesh,
        out_type=jax.ShapeDtypeStruct((XS_ROWS, D), jnp.float32),
        scratch_types=[
            pltpu.VMEM((K * CHT,), jnp.int32),
            pltpu.VMEM((K * CHT,), jnp.int32),
            pltpu.VMEM((K * CHT, D), jnp.float32),
            pltpu.SemaphoreType.DMA,
        ],
    )
    def _dispatch(x_hbm, poss_hbm, xs_hbm, dup_v, pos_v, rows_v, sem):
        wid = lax.axis_index("s") * NC + lax.axis_index("c")
        t0 = wid * TPW
        lane = lax.iota(jnp.int32, L)

        def chunk(ci, _):
            tc0 = t0 + ci * CHT
            dup_v[...] = tc0 + lax.shift_right_logical(lane, 1)
            pltpu.sync_copy(poss_hbm.at[pl.ds(tc0 * K, K * CHT)], pos_v)
            pltpu.async_copy(x_hbm.at[dup_v], rows_v, sem).wait()
            pltpu.async_copy(rows_v, xs_hbm.at[pos_v], sem).wait()
            return 0

        lax.fori_loop(0, NCH, chunk, 0)

    @functools.partial(
        pl.kernel,
        mesh=mesh,
        out_type=jax.ShapeDtypeStruct((T, D), jnp.float32),
        scratch_types=[
            pltpu.VMEM((K * CHT,), jnp.int32),
            pltpu.VMEM((K * CHT,), jnp.float32),
            pltpu.VMEM((K * CHT, D), jnp.float32),
            pltpu.VMEM((CHT, D), jnp.float32),
            pltpu.VMEM((CHT, D), jnp.float32),
            pltpu.SemaphoreType.DMA,
        ],
        compiler_params=pltpu.CompilerParams(needs_layout_passes=False),
    )
    def _combine(ys_hbm, yssh_hbm, posg_hbm, wts_hbm, out_hbm, pos_v, w_v,
                 rows_v, sh_v, o_v, sem):
        wid = lax.axis_index("s") * NC + lax.axis_index("c")
        t0 = wid * TPW

        zero16 = lax.iota(jnp.int32, L) * 0

        def chunk(ci, _):
            tc0 = t0 + ci * CHT
            pltpu.sync_copy(posg_hbm.at[pl.ds(tc0 * K, K * CHT)], pos_v)
            pltpu.sync_copy(wts_hbm.at[pl.ds(tc0 * K, K * CHT)], w_v)
            pltpu.async_copy(ys_hbm.at[pos_v], rows_v, sem).wait()
            pltpu.sync_copy(yssh_hbm.at[pl.ds(tc0, CHT)], sh_v)
            for j in range(CHT):
                w1 = plsc.load_gather(w_v, [zero16 + (K * j + 1)])
                if j == 0:
                    w0 = 1.0 - w1
                else:
                    w0 = plsc.load_gather(w_v, [zero16 + (K * j)])

                def col(v, _):
                    sl = pl.ds(v * L, L)
                    o_v[j, sl] = (sh_v[j, sl] + w0 * rows_v[K * j, sl]
                                  + w1 * rows_v[K * j + 1, sl])
                    return 0

                lax.fori_loop(0, D // L, col, 0)
            pltpu.sync_copy(o_v, out_hbm.at[pl.ds(tc0, CHT)])
            return 0

        lax.fori_loop(0, NCH, chunk, 0)

    return _dispatch, _combine


def kernel(hidden_states, gate_w, gate_bias, w_gate_up, w_down, ws_gate_up,
           ws_down):
    x = hidden_states
    gb = gate_bias.reshape(1, E)
    wgu = w_gate_up.reshape(E, 2, FF, D)
    wgus = ws_gate_up.reshape(1, 2, FF, D)
    wdn = w_down.reshape(E, D, FF)
    wdns = ws_down.reshape(1, D, FF)

    dispatch, combine = _sc_kernels()
    poss, posg, wts, cnts = _route(x, gate_w, gb)
    xs = dispatch(x, poss.reshape(T * K))
    ys_sh = _shared_mlp(x, wgus, wdns)
    ys = _mlp(cnts, xs, wgu, wdn)
    out = combine(ys, ys_sh, posg.reshape(T * K), wts.reshape(T * K))
    return out


# speedup vs baseline: 1.8191x; 1.0205x over previous
import functools

import jax
import jax.numpy as jnp
from jax import lax
from jax.experimental import pallas as pl
from jax.experimental.pallas import tpu as pltpu
from jax.experimental.pallas import tpu_sc as plsc

T = 2048
D = 2048
E = 16
K = 2
FF = 1408
NG = 8
TG = 4
CAP = 512

TB = 256
NTB = T // TB
SLOTS = E * CAP
XS_ROWS = SLOTS + CAP
YS_ROWS = SLOTS + T
NSB = T // CAP
NRB = E + NSB
FT = 128
NF = FF // FT
RC = 128
NRC = CAP // RC

NC, NS, L = 2, 16, 16
NW = NC * NS
TPW = T // NW
CHT = 8
NCH = TPW // CHT

_NEG = -float("inf")


def _routing_body(x_ref, gw_ref, gb_ref, poss_ref, posg_ref, wts_ref,
                  cnt_ref, carry_ref):
    tb = pl.program_id(0)

    @pl.when(tb == 0)
    def _():
        carry_ref[...] = jnp.zeros_like(carry_ref)

    xb = x_ref[...]
    logits = lax.dot_general(xb, gw_ref[...], (((1,), (1,)), ((), ())),
                             preferred_element_type=jnp.float32)
    scores = jax.nn.sigmoid(logits)
    sfc = scores + gb_ref[...]

    ri = lax.broadcasted_iota(jnp.int32, (E, NG), 0)
    ci = lax.broadcasted_iota(jnp.int32, (E, NG), 1)
    m1 = (ri // (E // NG) == ci).astype(jnp.float32)

    gs = lax.dot_general(sfc, m1, (((1,), (0,)), ((), ())),
                         precision=lax.Precision.HIGHEST,
                         preferred_element_type=jnp.float32)
    lane8 = lax.broadcasted_iota(jnp.int32, (TB, NG), 1)
    gmask = jnp.zeros((TB, NG), jnp.float32)
    for _ in range(TG):
        m = jnp.max(gs, axis=1, keepdims=True)
        amax = jnp.min(jnp.where(gs == m, lane8, NG), axis=1, keepdims=True)
        pick = lane8 == amax
        gmask = jnp.where(pick, 1.0, gmask)
        gs = jnp.where(pick, _NEG, gs)
    mask16 = lax.dot_general(gmask, m1, (((1,), (1,)), ((), ())),
                             preferred_element_type=jnp.float32)

    masked = jnp.where(mask16 > 0, sfc, _NEG)
    lane16 = lax.broadcasted_iota(jnp.int32, (TB, E), 1)
    idxs, ws, picks = [], [], []
    for _ in range(K):
        m = jnp.max(masked, axis=1, keepdims=True)
        amax = jnp.min(jnp.where(masked == m, lane16, E), axis=1, keepdims=True)
        pick = lane16 == amax
        idxs.append(amax)
        ws.append(jnp.sum(jnp.where(pick, scores, 0.0), axis=1, keepdims=True))
        picks.append(pick)
        masked = jnp.where(pick, _NEG, masked)
    denom = ws[0] + ws[1] + 1e-20
    w0 = ws[0] / denom
    w1 = ws[1] / denom

    cnt = picks[0].astype(jnp.float32) + picks[1].astype(jnp.float32)
    rr = lax.broadcasted_iota(jnp.int32, (TB, TB), 0)
    cc = lax.broadcasted_iota(jnp.int32, (TB, TB), 1)
    ltri = (cc < rr).astype(jnp.float32)
    excl = lax.dot_general(ltri, cnt, (((1,), (0,)), ((), ())),
                           preferred_element_type=jnp.float32) + carry_ref[...]
    carry_ref[...] = carry_ref[...] + jnp.sum(cnt, axis=0, keepdims=True)

    rank0 = jnp.sum(jnp.where(picks[0], excl, 0.0), axis=1,
                    keepdims=True).astype(jnp.int32)
    rank1 = jnp.sum(jnp.where(picks[1], excl, 0.0), axis=1,
                    keepdims=True).astype(jnp.int32)
    base0 = idxs[0] * CAP
    base1 = idxs[1] * CAP
    ok0 = rank0 < CAP
    ok1 = rank1 < CAP
    poss_ref[...] = jnp.concatenate(
        [jnp.where(ok0, base0 + rank0, SLOTS),
         jnp.where(ok1, base1 + rank1, SLOTS)], axis=1)
    posg_ref[...] = jnp.concatenate(
        [jnp.where(ok0, base0 + rank0, base0),
         jnp.where(ok1, base1 + rank1, base1)], axis=1)
    wts_ref[...] = jnp.concatenate(
        [jnp.where(ok0, w0, 0.0), jnp.where(ok1, w1, 0.0)], axis=1)

    @pl.when(tb == NTB - 1)
    def _():
        cnt_ref[...] = jnp.minimum(carry_ref[...], float(CAP)).astype(jnp.int32)


def _route(x, gate_w, gb):
    return pl.pallas_call(
        _routing_body,
        grid=(NTB,),
        in_specs=[
            pl.BlockSpec((TB, D), lambda tb: (tb, 0)),
            pl.BlockSpec((E, D), lambda tb: (0, 0)),
            pl.BlockSpec((1, E), lambda tb: (0, 0)),
        ],
        out_specs=[
            pl.BlockSpec((TB, K), lambda tb: (tb, 0)),
            pl.BlockSpec((TB, K), lambda tb: (tb, 0)),
            pl.BlockSpec((TB, K), lambda tb: (tb, 0)),
            pl.BlockSpec((1, E), lambda tb: (0, 0)),
        ],
        out_shape=[
            jax.ShapeDtypeStruct((T, K), jnp.int32),
            jax.ShapeDtypeStruct((T, K), jnp.int32),
            jax.ShapeDtypeStruct((T, K), jnp.float32),
            jax.ShapeDtypeStruct((1, E), jnp.int32),
        ],
        scratch_shapes=[pltpu.VMEM((1, E), jnp.float32)],
        compiler_params=pltpu.CompilerParams(
            dimension_semantics=("arbitrary",)),
    )(x, gate_w, gb)


def _ffn_step(nrows, x_in, wgu_in, wdn_in, ys_ref, h_ref, ft, nrc=NRC):
    def when_active(rc, fn):
        if nrows is None:
            fn()
        else:
            pl.when(rc * RC < nrows)(fn)

    wboth = wgu_in[...].reshape(2 * FT, D)
    for rc in range(nrc):
        sl = slice(rc * RC, (rc + 1) * RC)

        def gu_step(sl=sl):
            xv = x_in[sl, :]
            gu = lax.dot_general(xv, wboth, (((1,), (1,)), ((), ())),
                                 preferred_element_type=jnp.float32)
            g = gu[:, :FT]
            u = gu[:, FT:]
            h_ref[sl, pl.ds(ft * FT, FT)] = (g * jax.nn.sigmoid(g)) * u

        when_active(rc, gu_step)

    @pl.when(ft == NF - 1)
    def _():
        wd = wdn_in[0]
        for rc in range(nrc):
            sl = slice(rc * RC, (rc + 1) * RC)

            def dn_step(sl=sl):
                ys_ref[sl, :] = lax.dot_general(
                    h_ref[sl, :], wd, (((1,), (1,)), ((), ())),
                    preferred_element_type=jnp.float32)

            when_active(rc, dn_step)


def _mlp_body(cnts_ref, xs_ref, wgu_ref, wdn_ref, ys_ref, h_ref):
    rb = pl.program_id(0)
    ft = pl.program_id(1)
    nrows = cnts_ref[0, rb]
    _ffn_step(nrows, xs_ref, wgu_ref, wdn_ref, ys_ref, h_ref, ft)


def _mlp(cnts, xs, wgu, wdn):
    grid_spec = pltpu.PrefetchScalarGridSpec(
        num_scalar_prefetch=1,
        grid=(E, NF),
        in_specs=[
            pl.BlockSpec((CAP, D), lambda rb, ft, c: (rb, 0)),
            pl.BlockSpec((1, 2, FT, D), lambda rb, ft, c: (rb, 0, ft, 0)),
            pl.BlockSpec((1, D, FF), lambda rb, ft, c: (rb, 0, 0)),
        ],
        out_specs=pl.BlockSpec((CAP, D), lambda rb, ft, c: (rb, 0)),
        scratch_shapes=[pltpu.VMEM((CAP, FF), jnp.float32)],
    )
    return pl.pallas_call(
        _mlp_body,
        grid_spec=grid_spec,
        out_shape=jax.ShapeDtypeStruct((SLOTS, D), jnp.float32),
        compiler_params=pltpu.CompilerParams(
            dimension_semantics=("arbitrary", "arbitrary"),
            vmem_limit_bytes=60 * 1024 * 1024),
    )(cnts, xs, wgu, wdn)


SHB = T // 2


def _shared_body(x_ref, wgu_ref, wdn_ref, ys_ref, h_ref):
    ft = pl.program_id(0)
    _ffn_step(None, x_ref, wgu_ref, wdn_ref, ys_ref, h_ref, ft, nrc=SHB // RC)


def _shared_dep_body(x_ref, wgu_ref, wdn_ref, dep_ref, ys_ref, h_ref):
    del dep_ref
    _shared_body(x_ref, wgu_ref, wdn_ref, ys_ref, h_ref)


def _shared_mlp_half(x, wgus, wdns, half, dep=None):
    in_specs = [
        pl.BlockSpec((SHB, D), lambda ft: (half, 0)),
        pl.BlockSpec((1, 2, FT, D), lambda ft: (0, 0, ft, 0)),
        pl.BlockSpec((1, D, FF), lambda ft: (0, 0, 0)),
    ]
    operands = [x, wgus, wdns]
    body = _shared_body
    if dep is not None:
        in_specs.append(pl.BlockSpec((8, 128), lambda ft: (0, 0)))
        operands.append(dep)
        body = _shared_dep_body
    return pl.pallas_call(
        body,
        grid=(NF,),
        in_specs=in_specs,
        out_specs=pl.BlockSpec((SHB, D), lambda ft: (0, 0)),
        out_shape=jax.ShapeDtypeStruct((SHB, D), jnp.float32),
        scratch_shapes=[pltpu.VMEM((SHB, FF), jnp.float32)],
        compiler_params=pltpu.CompilerParams(
            dimension_semantics=("arbitrary",),
            vmem_limit_bytes=60 * 1024 * 1024),
    )(*operands)


def _add_body(tmp_ref, a_ref, b_ref, o_ref):
    h = pl.program_id(0)

    @pl.when(h == 0)
    def _():
        o_ref[...] = tmp_ref[...] + a_ref[...]

    @pl.when(h == 1)
    def _():
        o_ref[...] = tmp_ref[...] + b_ref[...]


def _add_shared(tmp, sha, shb):
    return pl.pallas_call(
        _add_body,
        grid=(2, 2),
        in_specs=[
            pl.BlockSpec((CAP, D), lambda h, s: (2 * h + s, 0)),
            pl.BlockSpec((CAP, D), lambda h, s: (s, 0)),
            pl.BlockSpec((CAP, D), lambda h, s: (s, 0)),
        ],
        out_specs=pl.BlockSpec((CAP, D), lambda h, s: (2 * h + s, 0)),
        out_shape=jax.ShapeDtypeStruct((T, D), jnp.float32),
        compiler_params=pltpu.CompilerParams(
            dimension_semantics=("arbitrary", "arbitrary")),
    )(tmp, sha, shb)


@functools.cache
def _sc_kernels():
    mesh = plsc.VectorSubcoreMesh(core_axis_name="c", subcore_axis_name="s",
                                  num_cores=NC, num_subcores=NS)

    @functools.partial(
        pl.kernel,
        mesh=mesh,
        out_type=jax.ShapeDtypeStruct((XS_ROWS, D), jnp.float32),
        scratch_types=[
            pltpu.VMEM((K * CHT,), jnp.int32),
            pltpu.VMEM((K * CHT,), jnp.int32),
            pltpu.VMEM((K * CHT, D), jnp.float32),
            pltpu.SemaphoreType.DMA,
        ],
    )
    def _dispatch(x_hbm, poss_hbm, xs_hbm, dup_v, pos_v, rows_v, sem):
        wid = lax.axis_index("s") * NC + lax.axis_index("c")
        t0 = wid * TPW
        lane = lax.iota(jnp.int32, L)

        def chunk(ci, _):
            tc0 = t0 + ci * CHT
            dup_v[...] = tc0 + lax.shift_right_logical(lane, 1)
            pltpu.sync_copy(poss_hbm.at[pl.ds(tc0 * K, K * CHT)], pos_v)
            pltpu.async_copy(x_hbm.at[dup_v], rows_v, sem).wait()
            pltpu.async_copy(rows_v, xs_hbm.at[pos_v], sem).wait()
            return 0

        lax.fori_loop(0, NCH, chunk, 0)

    @functools.partial(
        pl.kernel,
        mesh=mesh,
        out_type=jax.ShapeDtypeStruct((T, D), jnp.float32),
        scratch_types=[
            pltpu.VMEM((K * CHT,), jnp.int32),
            pltpu.VMEM((K * CHT,), jnp.float32),
            pltpu.VMEM((K * CHT, D), jnp.float32),
            pltpu.VMEM((CHT, D), jnp.float32),
            pltpu.SemaphoreType.DMA,
        ],
        compiler_params=pltpu.CompilerParams(needs_layout_passes=False),
    )
    def _combine(ys_hbm, posg_hbm, wts_hbm, out_hbm, pos_v, w_v,
                 rows_v, o_v, sem):
        wid = lax.axis_index("s") * NC + lax.axis_index("c")
        t0 = wid * TPW

        zero16 = lax.iota(jnp.int32, L) * 0

        def chunk(ci, _):
            tc0 = t0 + ci * CHT
            pltpu.sync_copy(posg_hbm.at[pl.ds(tc0 * K, K * CHT)], pos_v)
            pltpu.sync_copy(wts_hbm.at[pl.ds(tc0 * K, K * CHT)], w_v)
            pltpu.async_copy(ys_hbm.at[pos_v], rows_v, sem).wait()
            for j in range(CHT):
                w1 = plsc.load_gather(w_v, [zero16 + (K * j + 1)])
                if j == 0:
                    w0 = 1.0 - w1
                else:
                    w0 = plsc.load_gather(w_v, [zero16 + (K * j)])

                def col(v, _):
                    sl = pl.ds(v * L, L)
                    o_v[j, sl] = (w0 * rows_v[K * j, sl]
                                  + w1 * rows_v[K * j + 1, sl])
                    return 0

                lax.fori_loop(0, D // L, col, 0)
            pltpu.sync_copy(o_v, out_hbm.at[pl.ds(tc0, CHT)])
            return 0

        lax.fori_loop(0, NCH, chunk, 0)

    return _dispatch, _combine


def kernel(hidden_states, gate_w, gate_bias, w_gate_up, w_down, ws_gate_up,
           ws_down):
    x = hidden_states
    gb = gate_bias.reshape(1, E)
    wgu = w_gate_up.reshape(E, 2, FF, D)
    wgus = ws_gate_up.reshape(1, 2, FF, D)
    wdn = w_down.reshape(E, D, FF)
    wdns = ws_down.reshape(1, D, FF)

    dispatch, combine = _sc_kernels()
    poss, posg, wts, cnts = _route(x, gate_w, gb)
    xs = dispatch(x, poss.reshape(T * K))
    sh_a = _shared_mlp_half(x, wgus, wdns, 0)
    ys = _mlp(cnts, xs, wgu, wdn)
    sh_b = _shared_mlp_half(x, wgus, wdns, 1, dep=ys)
    tmp = combine(ys, posg.reshape(T * K), wts.reshape(T * K))
    return _add_shared(tmp, sh_a, sh_b)


# speedup vs baseline: 1.8542x; 1.0192x over previous
import functools

import jax
import jax.numpy as jnp
from jax import lax
from jax.experimental import pallas as pl
from jax.experimental.pallas import tpu as pltpu
from jax.experimental.pallas import tpu_sc as plsc

T = 2048
D = 2048
E = 16
K = 2
FF = 1408
NG = 8
TG = 4
CAP = 512

TB = 256
NTB = T // TB
SLOTS = E * CAP
XS_ROWS = SLOTS + CAP
YS_ROWS = SLOTS + T
NSB = T // CAP
NRB = E + NSB
FT = 128
NF = FF // FT
RC = 128
NRC = CAP // RC

NC, NS, L = 2, 16, 16
NW = NC * NS
TPW = T // NW
CHT = 8
NCH = TPW // CHT

_NEG = -float("inf")


def _routing_body(x_ref, gw_ref, gb_ref, poss_ref, posg_ref, wts_ref,
                  cnt_ref, carry_ref):
    tb = pl.program_id(0)

    @pl.when(tb == 0)
    def _():
        carry_ref[...] = jnp.zeros_like(carry_ref)

    xb = x_ref[...]
    logits = lax.dot_general(xb, gw_ref[...], (((1,), (1,)), ((), ())),
                             preferred_element_type=jnp.float32)
    scores = jax.nn.sigmoid(logits)
    sfc = scores + gb_ref[...]

    ri = lax.broadcasted_iota(jnp.int32, (E, NG), 0)
    ci = lax.broadcasted_iota(jnp.int32, (E, NG), 1)
    m1 = (ri // (E // NG) == ci).astype(jnp.float32)

    gs = lax.dot_general(sfc, m1, (((1,), (0,)), ((), ())),
                         precision=lax.Precision.HIGHEST,
                         preferred_element_type=jnp.float32)
    lane8 = lax.broadcasted_iota(jnp.int32, (TB, NG), 1)
    gmask = jnp.zeros((TB, NG), jnp.float32)
    for _ in range(TG):
        m = jnp.max(gs, axis=1, keepdims=True)
        amax = jnp.min(jnp.where(gs == m, lane8, NG), axis=1, keepdims=True)
        pick = lane8 == amax
        gmask = jnp.where(pick, 1.0, gmask)
        gs = jnp.where(pick, _NEG, gs)
    mask16 = lax.dot_general(gmask, m1, (((1,), (1,)), ((), ())),
                             preferred_element_type=jnp.float32)

    masked = jnp.where(mask16 > 0, sfc, _NEG)
    lane16 = lax.broadcasted_iota(jnp.int32, (TB, E), 1)
    idxs, ws, picks = [], [], []
    for _ in range(K):
        m = jnp.max(masked, axis=1, keepdims=True)
        amax = jnp.min(jnp.where(masked == m, lane16, E), axis=1, keepdims=True)
        pick = lane16 == amax
        idxs.append(amax)
        ws.append(jnp.sum(jnp.where(pick, scores, 0.0), axis=1, keepdims=True))
        picks.append(pick)
        masked = jnp.where(pick, _NEG, masked)
    denom = ws[0] + ws[1] + 1e-20
    w0 = ws[0] / denom
    w1 = ws[1] / denom

    cnt = picks[0].astype(jnp.float32) + picks[1].astype(jnp.float32)
    rr = lax.broadcasted_iota(jnp.int32, (TB, TB), 0)
    cc = lax.broadcasted_iota(jnp.int32, (TB, TB), 1)
    ltri = (cc < rr).astype(jnp.float32)
    excl = lax.dot_general(ltri, cnt, (((1,), (0,)), ((), ())),
                           preferred_element_type=jnp.float32) + carry_ref[...]
    carry_ref[...] = carry_ref[...] + jnp.sum(cnt, axis=0, keepdims=True)

    rank0 = jnp.sum(jnp.where(picks[0], excl, 0.0), axis=1,
                    keepdims=True).astype(jnp.int32)
    rank1 = jnp.sum(jnp.where(picks[1], excl, 0.0), axis=1,
                    keepdims=True).astype(jnp.int32)
    base0 = idxs[0] * CAP
    base1 = idxs[1] * CAP
    ok0 = rank0 < CAP
    ok1 = rank1 < CAP
    poss_ref[...] = jnp.concatenate(
        [jnp.where(ok0, base0 + rank0, SLOTS),
         jnp.where(ok1, base1 + rank1, SLOTS)], axis=1)
    posg_ref[...] = jnp.concatenate(
        [jnp.where(ok0, base0 + rank0, base0),
         jnp.where(ok1, base1 + rank1, base1)], axis=1)
    wts_ref[...] = jnp.concatenate(
        [jnp.where(ok0, w0, 0.0), jnp.where(ok1, w1, 0.0)], axis=1)

    @pl.when(tb == NTB - 1)
    def _():
        cnt_ref[...] = jnp.minimum(carry_ref[...], float(CAP)).astype(jnp.int32)


def _route(x, gate_w, gb):
    return pl.pallas_call(
        _routing_body,
        grid=(NTB,),
        in_specs=[
            pl.BlockSpec((TB, D), lambda tb: (tb, 0)),
            pl.BlockSpec((E, D), lambda tb: (0, 0)),
            pl.BlockSpec((1, E), lambda tb: (0, 0)),
        ],
        out_specs=[
            pl.BlockSpec((TB, K), lambda tb: (tb, 0)),
            pl.BlockSpec((TB, K), lambda tb: (tb, 0)),
            pl.BlockSpec((TB, K), lambda tb: (tb, 0)),
            pl.BlockSpec((1, E), lambda tb: (0, 0)),
        ],
        out_shape=[
            jax.ShapeDtypeStruct((T, K), jnp.int32),
            jax.ShapeDtypeStruct((T, K), jnp.int32),
            jax.ShapeDtypeStruct((T, K), jnp.float32),
            jax.ShapeDtypeStruct((1, E), jnp.int32),
        ],
        scratch_shapes=[pltpu.VMEM((1, E), jnp.float32)],
        compiler_params=pltpu.CompilerParams(
            dimension_semantics=("arbitrary",)),
    )(x, gate_w, gb)


def _ffn_step(nrows, x_in, wgu_in, wdn_in, ys_ref, h_ref, ft, nrc=NRC):
    def when_active(rc, fn):
        if nrows is None:
            fn()
        else:
            pl.when(rc * RC < nrows)(fn)

    wboth = wgu_in[...].reshape(2 * FT, D)
    for rc in range(nrc):
        sl = slice(rc * RC, (rc + 1) * RC)

        def gu_step(sl=sl):
            xv = x_in[sl, :]
            gu = lax.dot_general(xv, wboth, (((1,), (1,)), ((), ())),
                                 preferred_element_type=jnp.float32)
            g = gu[:, :FT]
            u = gu[:, FT:]
            h_ref[sl, pl.ds(ft * FT, FT)] = (g * jax.nn.sigmoid(g)) * u

        when_active(rc, gu_step)

    @pl.when(ft == NF - 1)
    def _():
        wd = wdn_in[0]
        for rc in range(nrc):
            sl = slice(rc * RC, (rc + 1) * RC)

            def dn_step(sl=sl):
                ys_ref[sl, :] = lax.dot_general(
                    h_ref[sl, :], wd, (((1,), (1,)), ((), ())),
                    preferred_element_type=jnp.float32)

            when_active(rc, dn_step)


def _mlp_body(cnts_ref, xs_ref, wgu_ref, wdn_ref, dep_ref, ys_ref, h_ref):
    del dep_ref
    rb = pl.program_id(0)
    ft = pl.program_id(1)
    nrows = cnts_ref[0, rb]
    _ffn_step(nrows, xs_ref, wgu_ref, wdn_ref, ys_ref, h_ref, ft)


def _mlp(cnts, xs, wgu, wdn, dep):
    grid_spec = pltpu.PrefetchScalarGridSpec(
        num_scalar_prefetch=1,
        grid=(E, NF),
        in_specs=[
            pl.BlockSpec((CAP, D), lambda rb, ft, c: (rb, 0)),
            pl.BlockSpec((1, 2, FT, D), lambda rb, ft, c: (rb, 0, ft, 0)),
            pl.BlockSpec((1, D, FF), lambda rb, ft, c: (rb, 0, 0)),
            pl.BlockSpec((8, 128), lambda rb, ft, c: (0, 0)),
        ],
        out_specs=pl.BlockSpec((CAP, D), lambda rb, ft, c: (rb, 0)),
        scratch_shapes=[pltpu.VMEM((CAP, FF), jnp.float32)],
    )
    return pl.pallas_call(
        _mlp_body,
        grid_spec=grid_spec,
        out_shape=jax.ShapeDtypeStruct((SLOTS, D), jnp.float32),
        compiler_params=pltpu.CompilerParams(
            dimension_semantics=("arbitrary", "arbitrary"),
            vmem_limit_bytes=60 * 1024 * 1024),
    )(cnts, xs, wgu, wdn, dep)


SHB = T // 2


def _shared_body(x_ref, wgu_ref, wdn_ref, ys_ref, h_ref):
    ft = pl.program_id(0)
    _ffn_step(None, x_ref, wgu_ref, wdn_ref, ys_ref, h_ref, ft, nrc=SHB // RC)


def _shared_dep_body(x_ref, wgu_ref, wdn_ref, dep_ref, ys_ref, h_ref):
    del dep_ref
    _shared_body(x_ref, wgu_ref, wdn_ref, ys_ref, h_ref)


def _shared_mlp_half(x, wgus, wdns, half, dep=None):
    in_specs = [
        pl.BlockSpec((SHB, D), lambda ft: (half, 0)),
        pl.BlockSpec((1, 2, FT, D), lambda ft: (0, 0, ft, 0)),
        pl.BlockSpec((1, D, FF), lambda ft: (0, 0, 0)),
    ]
    operands = [x, wgus, wdns]
    body = _shared_body
    if dep is not None:
        in_specs.append(pl.BlockSpec((8, 128), lambda ft: (0, 0)))
        operands.append(dep)
        body = _shared_dep_body
    return pl.pallas_call(
        body,
        grid=(NF,),
        in_specs=in_specs,
        out_specs=pl.BlockSpec((SHB, D), lambda ft: (0, 0)),
        out_shape=jax.ShapeDtypeStruct((SHB, D), jnp.float32),
        scratch_shapes=[pltpu.VMEM((SHB, FF), jnp.float32)],
        compiler_params=pltpu.CompilerParams(
            dimension_semantics=("arbitrary",),
            vmem_limit_bytes=60 * 1024 * 1024),
    )(*operands)


def _add_body(tmp_ref, a_ref, b_ref, o_ref):
    h = pl.program_id(0)

    @pl.when(h == 0)
    def _():
        o_ref[...] = tmp_ref[...] + a_ref[...]

    @pl.when(h == 1)
    def _():
        o_ref[...] = tmp_ref[...] + b_ref[...]


def _add_shared(tmp, sha, shb):
    return pl.pallas_call(
        _add_body,
        grid=(2, 2),
        in_specs=[
            pl.BlockSpec((CAP, D), lambda h, s: (2 * h + s, 0)),
            pl.BlockSpec((CAP, D), lambda h, s: (s, 0)),
            pl.BlockSpec((CAP, D), lambda h, s: (s, 0)),
        ],
        out_specs=pl.BlockSpec((CAP, D), lambda h, s: (2 * h + s, 0)),
        out_shape=jax.ShapeDtypeStruct((T, D), jnp.float32),
        compiler_params=pltpu.CompilerParams(
            dimension_semantics=("arbitrary", "arbitrary")),
    )(tmp, sha, shb)


@functools.cache
def _sc_kernels():
    mesh = plsc.VectorSubcoreMesh(core_axis_name="c", subcore_axis_name="s",
                                  num_cores=NC, num_subcores=NS)

    @functools.partial(
        pl.kernel,
        mesh=mesh,
        out_type=jax.ShapeDtypeStruct((XS_ROWS, D), jnp.float32),
        scratch_types=[
            pltpu.VMEM((K * CHT,), jnp.int32),
            pltpu.VMEM((K * CHT,), jnp.int32),
            pltpu.VMEM((K * CHT, D), jnp.float32),
            pltpu.SemaphoreType.DMA,
        ],
    )
    def _dispatch(x_hbm, poss_hbm, xs_hbm, dup_v, pos_v, rows_v, sem):
        wid = lax.axis_index("s") * NC + lax.axis_index("c")
        t0 = wid * TPW
        lane = lax.iota(jnp.int32, L)

        def chunk(ci, _):
            tc0 = t0 + ci * CHT
            dup_v[...] = tc0 + lax.shift_right_logical(lane, 1)
            pltpu.sync_copy(poss_hbm.at[pl.ds(tc0 * K, K * CHT)], pos_v)
            pltpu.async_copy(x_hbm.at[dup_v], rows_v, sem).wait()
            pltpu.async_copy(rows_v, xs_hbm.at[pos_v], sem).wait()
            return 0

        lax.fori_loop(0, NCH, chunk, 0)

    @functools.partial(
        pl.kernel,
        mesh=mesh,
        out_type=jax.ShapeDtypeStruct((T, D), jnp.float32),
        scratch_types=[
            pltpu.VMEM((K * CHT,), jnp.int32),
            pltpu.VMEM((K * CHT,), jnp.float32),
            pltpu.VMEM((K * CHT, D), jnp.float32),
            pltpu.VMEM((CHT, D), jnp.float32),
            pltpu.SemaphoreType.DMA,
        ],
        compiler_params=pltpu.CompilerParams(needs_layout_passes=False),
    )
    def _combine(ys_hbm, posg_hbm, wts_hbm, out_hbm, pos_v, w_v,
                 rows_v, o_v, sem):
        wid = lax.axis_index("s") * NC + lax.axis_index("c")
        t0 = wid * TPW

        zero16 = lax.iota(jnp.int32, L) * 0

        def chunk(ci, _):
            tc0 = t0 + ci * CHT
            pltpu.sync_copy(posg_hbm.at[pl.ds(tc0 * K, K * CHT)], pos_v)
            pltpu.sync_copy(wts_hbm.at[pl.ds(tc0 * K, K * CHT)], w_v)
            pltpu.async_copy(ys_hbm.at[pos_v], rows_v, sem).wait()
            for j in range(CHT):
                w1 = plsc.load_gather(w_v, [zero16 + (K * j + 1)])
                if j == 0:
                    w0 = 1.0 - w1
                else:
                    w0 = plsc.load_gather(w_v, [zero16 + (K * j)])

                def col(v, _):
                    sl = pl.ds(v * L, L)
                    o_v[j, sl] = (w0 * rows_v[K * j, sl]
                                  + w1 * rows_v[K * j + 1, sl])
                    return 0

                lax.fori_loop(0, D // L, col, 0)
            pltpu.sync_copy(o_v, out_hbm.at[pl.ds(tc0, CHT)])
            return 0

        lax.fori_loop(0, NCH, chunk, 0)

    return _dispatch, _combine


def kernel(hidden_states, gate_w, gate_bias, w_gate_up, w_down, ws_gate_up,
           ws_down):
    x = hidden_states
    gb = gate_bias.reshape(1, E)
    wgu = w_gate_up.reshape(E, 2, FF, D)
    wgus = ws_gate_up.reshape(1, 2, FF, D)
    wdn = w_down.reshape(E, D, FF)
    wdns = ws_down.reshape(1, D, FF)

    dispatch, combine = _sc_kernels()
    poss, posg, wts, cnts = _route(x, gate_w, gb)
    xs = dispatch(x, poss.reshape(T * K))
    sh_a = _shared_mlp_half(x, wgus, wdns, 0)
    ys = _mlp(cnts, xs, wgu, wdn, sh_a)
    sh_b = _shared_mlp_half(x, wgus, wdns, 1, dep=ys)
    tmp = combine(ys, posg.reshape(T * K), wts.reshape(T * K))
    return _add_shared(tmp, sh_a, sh_b)
